# baseline (speedup 1.0000x reference)
import functools

import jax
import jax.numpy as jnp
from jax import lax
from jax.experimental import pallas as pl
from jax.experimental.pallas import tpu as pltpu
from jax.experimental.pallas import tpu_sc as plsc

N = 10000
D = 128
E = 320000

NC = 2
NS = 16
NW = NC * NS
B = 80
NB = 125
CH = 5
NCH = NB // CH
E_PAD = NW * NB * B
KBUF = 4

N_PAD = 10240
ROWS_PER_TILE = N_PAD // NS
ZCHUNK = 80
OCHUNK = 128


def _sc_kernel_body(x_hbm, src_hbm, dst_hbm, w_hbm, out_hbm,
                    src_r, dst_r, w_r, rows_v, acc, gsem, ssem, esem):
    c = lax.axis_index("c")
    s = lax.axis_index("s")
    wid = s * NC + c

    def eslot(jj):
        return lax.rem(lax.div(jj, CH), 2)

    def jrow(jj):
        return lax.rem(jj, CH)

    def src_ref(jj):
        return src_r.at[eslot(jj)].at[jrow(jj)]

    def dst_ref(jj):
        return dst_r.at[eslot(jj)].at[jrow(jj)]

    def echunk_start(cc, slot):
        pltpu.async_copy(src_hbm.at[wid].at[cc], src_r.at[slot], esem)
        pltpu.async_copy(dst_hbm.at[wid].at[cc], dst_r.at[slot], esem)
        pltpu.async_copy(w_hbm.at[wid].at[cc], w_r.at[slot], esem)

    def echunk_wait(cc, slot):
        pltpu.make_async_copy(src_hbm.at[wid].at[cc], src_r.at[slot],
                              esem).wait()
        pltpu.make_async_copy(dst_hbm.at[wid].at[cc], dst_r.at[slot],
                              esem).wait()
        pltpu.make_async_copy(w_hbm.at[wid].at[cc], w_r.at[slot],
                              esem).wait()

    def gather_start(jj, b):
        pltpu.async_copy(x_hbm.at[src_ref(jj)], rows_v.at[b], gsem.at[b])

    def gather_wait(jj, b):
        pltpu.make_async_copy(x_hbm.at[src_ref(jj)], rows_v.at[b],
                              gsem.at[b]).wait()

    def scatter_start(jj, b):
        pltpu.async_copy(rows_v.at[b], acc.at[dst_ref(jj)], ssem.at[b],
                         add=True)

    def scatter_wait(jj, b):
        pltpu.make_async_copy(rows_v.at[b], acc.at[dst_ref(jj)],
                              ssem.at[b]).wait()

    zero = jnp.zeros((16,), jnp.float32)

    def zrow(i, _):
        for cc in range(8):
            rows_v[0, i, pl.ds(cc * 16, 16)] = zero
        return ()

    lax.fori_loop(0, B, zrow, ())
    for k in range(ROWS_PER_TILE // ZCHUNK):
        pltpu.sync_copy(rows_v.at[0].at[pl.ds(0, ZCHUNK)],
                        acc.at[pl.ds(s * ROWS_PER_TILE + k * ZCHUNK, ZCHUNK)])
    plsc.subcore_barrier()

    echunk_start(0, 0)
    echunk_wait(0, 0)
    gather_start(0, 0)
    gather_start(1, 1)
    gather_start(2, 2)

    def block(j, _):
        b = lax.rem(j, KBUF)
        nb = lax.rem(j + 3, KBUF)
        cc = lax.div(j, CH)
        jr = lax.rem(j, CH)
        es = lax.rem(cc, 2)

        gather_wait(j, b)
        for g in range(B // 16):
            wvec = w_r[es, jr, pl.ds(g * 16, 16)]
            for e in range(16):
                r = g * 16 + e
                wb = jnp.broadcast_to(wvec[e], (16,))
                for col in range(8):
                    rows_v[b, r, pl.ds(col * 16, 16)] = (
                        rows_v[b, r, pl.ds(col * 16, 16)] * wb)
        scatter_start(j, b)

        @pl.when(j >= 1)
        def _():
            scatter_wait(j - 1, nb)

        @pl.when(jnp.logical_and(jr == 0, cc + 1 < NCH))
        def _():
            echunk_start(cc + 1, lax.rem(cc + 1, 2))

        @pl.when(jnp.logical_and(jr == CH - 3, cc + 1 < NCH))
        def _():
            echunk_wait(cc + 1, lax.rem(cc + 1, 2))

        @pl.when(j + 3 < NB)
        def _():
            gather_start(j + 3, nb)

        return ()

    lax.fori_loop(0, NB, block, ())
    scatter_wait(NB - 1, lax.rem(NB - 1, KBUF))
    plsc.subcore_barrier()

    for k in range(ROWS_PER_TILE // OCHUNK):
        base = s * ROWS_PER_TILE + k * OCHUNK
        pltpu.sync_copy(acc.at[pl.ds(base, OCHUNK)],
                        out_hbm.at[c].at[pl.ds(base, OCHUNK)])


@jax.jit
def _sc_aggregate(x, src4, dst4, w4):
    mesh = plsc.VectorSubcoreMesh(core_axis_name="c", subcore_axis_name="s")
    return pl.kernel(
        _sc_kernel_body,
        out_type=jax.ShapeDtypeStruct((NC, N_PAD, D), jnp.float32),
        mesh=mesh,
        scratch_types=[
            pltpu.VMEM((2, CH, B), jnp.int32),
            pltpu.VMEM((2, CH, B), jnp.int32),
            pltpu.VMEM((2, CH, B), jnp.float32),
            pltpu.VMEM((KBUF, B, D), jnp.float32),
            pltpu.VMEM_SHARED((N_PAD, D), jnp.float32),
            pltpu.SemaphoreType.DMA((KBUF,)),
            pltpu.SemaphoreType.DMA((KBUF,)),
            pltpu.SemaphoreType.DMA,
        ],
    )(x, src4, dst4, w4)


def _combine_body(p_ref, o_ref):
    o_ref[...] = p_ref[0] + p_ref[1]


@jax.jit
def _combine(partial):
    blk = 200
    return pl.pallas_call(
        _combine_body,
        out_shape=jax.ShapeDtypeStruct((N, D), jnp.float32),
        grid=(N // blk,),
        in_specs=[pl.BlockSpec((NC, blk, D), lambda i: (0, i, 0))],
        out_specs=pl.BlockSpec((blk, D), lambda i: (i, 0)),
    )(partial)


def kernel(x, edge_index, edge_weight):
    src = edge_index[0]
    dst = edge_index[1]
    src4 = src.reshape(NW, NCH, CH, B)
    dst4 = dst.reshape(NW, NCH, CH, B)
    w4 = edge_weight.reshape(NW, NCH, CH, B)
    partial = _sc_aggregate(x, src4, dst4, w4)
    return _combine(partial)


# speedup vs baseline: 1.0026x; 1.0026x over previous
import functools

import jax
import jax.numpy as jnp
from jax import lax
from jax.experimental import pallas as pl
from jax.experimental.pallas import tpu as pltpu
from jax.experimental.pallas import tpu_sc as plsc

N = 10000
D = 128
E = 320000

NC = 2
NS = 16
NW = NC * NS
B = 80
NB = 125
CH = 5
NCH = NB // CH
E_PAD = NW * NB * B
KBUF = 4

N_PAD = 10240
ROWS_PER_TILE = N_PAD // NS
ZCHUNK = 80
OCHUNK = 128


def _sc_kernel_body(x_hbm, src_hbm, dst_hbm, w_hbm, out_hbm,
                    src_r, dst_r, w_r, rows_v, acc, gsem, ssem, esem):
    c = lax.axis_index("c")
    s = lax.axis_index("s")
    wid = s * NC + c

    def eslot(jj):
        return lax.rem(lax.div(jj, CH), 2)

    def jrow(jj):
        return lax.rem(jj, CH)

    def src_ref(jj):
        return src_r.at[eslot(jj)].at[jrow(jj)]

    def dst_ref(jj):
        return dst_r.at[eslot(jj)].at[jrow(jj)]

    def echunk_start(cc, slot):
        pltpu.async_copy(src_hbm.at[wid].at[cc], src_r.at[slot], esem)
        pltpu.async_copy(dst_hbm.at[wid].at[cc], dst_r.at[slot], esem)
        pltpu.async_copy(w_hbm.at[wid].at[cc], w_r.at[slot], esem)

    def echunk_wait(cc, slot):
        pltpu.make_async_copy(src_hbm.at[wid].at[cc], src_r.at[slot],
                              esem).wait()
        pltpu.make_async_copy(dst_hbm.at[wid].at[cc], dst_r.at[slot],
                              esem).wait()
        pltpu.make_async_copy(w_hbm.at[wid].at[cc], w_r.at[slot],
                              esem).wait()

    def gather_start(jj, b):
        pltpu.async_copy(x_hbm.at[src_ref(jj)], rows_v.at[b], gsem.at[b])

    def gather_wait(jj, b):
        pltpu.make_async_copy(x_hbm.at[src_ref(jj)], rows_v.at[b],
                              gsem.at[b]).wait()

    def scatter_start(jj, b):
        pltpu.async_copy(rows_v.at[b], acc.at[dst_ref(jj)], ssem.at[b],
                         add=True)

    def scatter_wait(jj, b):
        pltpu.make_async_copy(rows_v.at[b], acc.at[dst_ref(jj)],
                              ssem.at[b]).wait()

    zero = jnp.zeros((16,), jnp.float32)

    def zrow(i, _):
        for cc in range(8):
            rows_v[0, i, pl.ds(cc * 16, 16)] = zero
        return ()

    lax.fori_loop(0, B, zrow, ())
    for k in range(ROWS_PER_TILE // ZCHUNK):
        pltpu.sync_copy(rows_v.at[0].at[pl.ds(0, ZCHUNK)],
                        acc.at[pl.ds(s * ROWS_PER_TILE + k * ZCHUNK, ZCHUNK)])
    plsc.subcore_barrier()

    echunk_start(0, 0)
    echunk_wait(0, 0)
    gather_start(0, 0)
    gather_start(1, 1)
    gather_start(2, 2)

    def block(j, _):
        b = lax.rem(j, KBUF)
        nb = lax.rem(j + 3, KBUF)
        cc = lax.div(j, CH)
        jr = lax.rem(j, CH)
        es = lax.rem(cc, 2)

        gather_wait(j, b)
        for g in range(B // 16):
            wvec = w_r[es, jr, pl.ds(g * 16, 16)]
            for e in range(16):
                r = g * 16 + e
                wb = jnp.broadcast_to(wvec[e], (16,))
                for col in range(8):
                    rows_v[b, r, pl.ds(col * 16, 16)] = (
                        rows_v[b, r, pl.ds(col * 16, 16)] * wb)
        scatter_start(j, b)

        @pl.when(j >= 1)
        def _():
            scatter_wait(j - 1, nb)

        @pl.when(jnp.logical_and(jr == 0, cc + 1 < NCH))
        def _():
            echunk_start(cc + 1, lax.rem(cc + 1, 2))

        @pl.when(jnp.logical_and(jr == CH - 3, cc + 1 < NCH))
        def _():
            echunk_wait(cc + 1, lax.rem(cc + 1, 2))

        @pl.when(j + 3 < NB)
        def _():
            gather_start(j + 3, nb)

        return ()

    lax.fori_loop(0, NB, block, ())
    scatter_wait(NB - 1, lax.rem(NB - 1, KBUF))
    plsc.subcore_barrier()

    for k in range(ROWS_PER_TILE // OCHUNK):
        base = s * ROWS_PER_TILE + k * OCHUNK
        pltpu.sync_copy(acc.at[pl.ds(base, OCHUNK)],
                        out_hbm.at[c].at[pl.ds(base, OCHUNK)])


def _sc_aggregate(x, src4, dst4, w4):
    mesh = plsc.VectorSubcoreMesh(core_axis_name="c", subcore_axis_name="s")
    return pl.kernel(
        _sc_kernel_body,
        out_type=jax.ShapeDtypeStruct((NC, N_PAD, D), jnp.float32),
        mesh=mesh,
        scratch_types=[
            pltpu.VMEM((2, CH, B), jnp.int32),
            pltpu.VMEM((2, CH, B), jnp.int32),
            pltpu.VMEM((2, CH, B), jnp.float32),
            pltpu.VMEM((KBUF, B, D), jnp.float32),
            pltpu.VMEM_SHARED((N_PAD, D), jnp.float32),
            pltpu.SemaphoreType.DMA((KBUF,)),
            pltpu.SemaphoreType.DMA((KBUF,)),
            pltpu.SemaphoreType.DMA,
        ],
    )(x, src4, dst4, w4)


def _combine_body(p_ref, o_ref):
    o_ref[...] = p_ref[0] + p_ref[1]


def _combine(partial):
    blk = 200
    return pl.pallas_call(
        _combine_body,
        out_shape=jax.ShapeDtypeStruct((N, D), jnp.float32),
        grid=(N // blk,),
        in_specs=[pl.BlockSpec((NC, blk, D), lambda i: (0, i, 0))],
        out_specs=pl.BlockSpec((blk, D), lambda i: (i, 0)),
    )(partial)


@jax.jit
def kernel(x, edge_index, edge_weight):
    src = edge_index[0]
    dst = edge_index[1]
    src4 = src.reshape(NW, NCH, CH, B)
    dst4 = dst.reshape(NW, NCH, CH, B)
    w4 = edge_weight.reshape(NW, NCH, CH, B)
    partial = _sc_aggregate(x, src4, dst4, w4)
    return _combine(partial)
